# Initial kernel scaffold; baseline (speedup 1.0000x reference)
import jax
import jax.numpy as jnp
from jax.experimental import pallas as pl


def kernel(x_author, x_paper, Wp_author, bp_author, Wp_paper, bp_paper, as_writes, ad_writes, as_wb, ad_wb, as_cites, ad_cites, Wk, bk, q, Wlin, blin, ei_writes, ei_written_by, ei_cites):
    raise NotImplementedError("write your pallas kernel here")


# speedup vs baseline: 32.4357x; 32.4357x over previous
import jax
import jax.numpy as jnp
from jax import lax
from jax.experimental import pallas as pl
from jax.experimental.pallas import tpu as pltpu
from jax.experimental.pallas import tpu_sc as plsc

N_NODE = 10000
D_IN = 256
HIDDEN = 128
HEADS = 8
HEAD_DIM = 16
OUT = 64
E = 160000
NEG_SLOPE = 0.2

NACC = 10240
NSEG = 10016
EP = 163840
CHUNK = 128
CHUNKS_PER_TILE = EP // (32 * CHUNK)
ROWS_PER_TILE = NSEG // 16
MROW = HIDDEN + 16


def _head_expand_mat(cols):
    r = lax.broadcasted_iota(jnp.int32, (HIDDEN, cols), 0)
    c = lax.broadcasted_iota(jnp.int32, (HIDDEN, cols), 1)
    return ((r // HEAD_DIM) == c).astype(jnp.float32)


def _dense_prep_body(xa_ref, xp_ref, Wpa_ref, Wpp_ref, bpa_ref, bpp_ref,
                     asw_ref, adw_ref, aswb_ref, adwb_ref, asc_ref, adc_ref,
                     Xa_ref, Xp_ref, Sw_ref, Dw_ref, Swb_ref, Dwb_ref,
                     Sc_ref, Dc_ref):
    xa = jnp.dot(xa_ref[:], Wpa_ref[:], preferred_element_type=jnp.float32)
    xa = xa + bpa_ref[:]
    xp = jnp.dot(xp_ref[:], Wpp_ref[:], preferred_element_type=jnp.float32)
    xp = xp + bpp_ref[:]
    G = _head_expand_mat(16)
    Xa_ref[:] = xa
    Xp_ref[:] = xp
    Sw_ref[:] = jnp.dot(xa * asw_ref[:], G, preferred_element_type=jnp.float32)
    Dw_ref[:] = jnp.dot(xp * adw_ref[:], G, preferred_element_type=jnp.float32)
    Swb_ref[:] = jnp.dot(xp * aswb_ref[:], G,
                         preferred_element_type=jnp.float32)
    Dwb_ref[:] = jnp.dot(xa * adwb_ref[:], G,
                         preferred_element_type=jnp.float32)
    Sc_ref[:] = jnp.dot(xp * asc_ref[:], G, preferred_element_type=jnp.float32)
    Dc_ref[:] = jnp.dot(xp * adc_ref[:], G, preferred_element_type=jnp.float32)


def _dense_prep(xa_pad, xp_pad, Wpa, Wpp, bpa, bpp, avecs):
    RB = 256
    grid = (NACC // RB,)
    full128 = pl.BlockSpec((D_IN, HIDDEN), lambda i: (0, 0))
    row128 = pl.BlockSpec((1, HIDDEN), lambda i: (0, 0))
    xblk = pl.BlockSpec((RB, D_IN), lambda i: (i, 0))
    o128 = pl.BlockSpec((RB, HIDDEN), lambda i: (i, 0))
    o16 = pl.BlockSpec((RB, 16), lambda i: (i, 0))
    out_shapes = (
        jax.ShapeDtypeStruct((NACC, HIDDEN), jnp.float32),
        jax.ShapeDtypeStruct((NACC, HIDDEN), jnp.float32),
    ) + tuple(jax.ShapeDtypeStruct((NACC, 16), jnp.float32)
              for _ in range(6))
    return pl.pallas_call(
        _dense_prep_body,
        grid=grid,
        in_specs=[xblk, xblk, full128, full128, row128, row128] + [row128] * 6,
        out_specs=(o128, o128) + (o16,) * 6,
        out_shape=out_shapes,
    )(xa_pad, xp_pad, Wpa, Wpp, bpa, bpp, *avecs)


def _edge_conv_body(si_hbm, di_hbm, X_hbm, S_hbm, D_hbm, z_hbm, out_hbm,
                    si, di, xrows, srows, drows, msg, acc, sem):
    c = lax.axis_index("c")
    s = lax.axis_index("s")
    wid = c * 16 + s
    pltpu.sync_copy(z_hbm.at[pl.ds(s * ROWS_PER_TILE, ROWS_PER_TILE)],
                    acc.at[pl.ds(s * ROWS_PER_TILE, ROWS_PER_TILE)])
    plsc.subcore_barrier()

    def chunk(j, carry):
        base = (wid * CHUNKS_PER_TILE + j) * CHUNK
        pltpu.sync_copy(si_hbm.at[pl.ds(base, CHUNK)], si)
        pltpu.sync_copy(di_hbm.at[pl.ds(base, CHUNK)], di)
        cp1 = pltpu.async_copy(X_hbm.at[si], xrows, sem)
        cp2 = pltpu.async_copy(S_hbm.at[si], srows, sem)
        cp3 = pltpu.async_copy(D_hbm.at[di], drows, sem)
        cp1.wait()
        cp2.wait()
        cp3.wait()

        def edge(e, carry2):
            a = srows[e, :] + drows[e, :]
            a = jnp.where(a > 0.0, a, NEG_SLOPE * a)
            w = jnp.exp(a)
            msg[e, pl.ds(HIDDEN, 16)] = w
            dn = lax.GatherDimensionNumbers(offset_dims=(),
                                            collapsed_slice_dims=(0,),
                                            start_index_map=(0,))
            for h in range(HEADS):
                sh = lax.gather(w, jnp.full((16, 1), h, jnp.int32), dn,
                                slice_sizes=(1,),
                                mode=lax.GatherScatterMode.PROMISE_IN_BOUNDS)
                msg[e, pl.ds(h * 16, 16)] = sh * xrows[e, pl.ds(h * 16, 16)]
            return carry2

        lax.fori_loop(0, CHUNK, edge, 0)
        pltpu.sync_copy(msg, acc.at[di], add=True)
        return carry

    lax.fori_loop(0, CHUNKS_PER_TILE, chunk, 0)
    plsc.subcore_barrier()
    pltpu.sync_copy(acc.at[pl.ds(s * ROWS_PER_TILE, ROWS_PER_TILE)],
                    out_hbm.at[c, pl.ds(s * ROWS_PER_TILE, ROWS_PER_TILE)])


def _edge_conv(src_idx, dst_idx, X, S, D, zeros):
    mesh = plsc.VectorSubcoreMesh(core_axis_name="c", subcore_axis_name="s",
                                  num_cores=2, num_subcores=16)
    f = pl.kernel(
        _edge_conv_body,
        out_type=jax.ShapeDtypeStruct((2, NSEG, MROW), jnp.float32),
        mesh=mesh,
        scratch_types=[
            pltpu.VMEM((CHUNK,), jnp.int32),
            pltpu.VMEM((CHUNK,), jnp.int32),
            pltpu.VMEM((CHUNK, HIDDEN), jnp.float32),
            pltpu.VMEM((CHUNK, 16), jnp.float32),
            pltpu.VMEM((CHUNK, 16), jnp.float32),
            pltpu.VMEM((CHUNK, MROW), jnp.float32),
            pltpu.VMEM_SHARED((NSEG, MROW), jnp.float32),
            pltpu.SemaphoreType.DMA,
        ],
        compiler_params=pltpu.CompilerParams(use_tc_tiling_on_sc=False),
    )
    return f(src_idx, dst_idx, X, S, D, zeros)


def _post_body(accw_ref, accc_ref, accwb_ref, Wk_ref, bk_ref, Wlin_ref,
               blin_ref, outw_ref, outc_ref, hauth_ref, ksw_ref, ksc_ref):
    i = pl.program_id(0)
    B = _head_expand_mat(16)

    def norm(acc_ref):
        sblk = acc_ref[0] + acc_ref[1]
        num = sblk[:, :HIDDEN]
        den = sblk[:, HIDDEN:HIDDEN + 16]
        den_b = jnp.dot(den, B.T, preferred_element_type=jnp.float32)
        return jnp.maximum(num / (den_b + 1e-16), 0.0)

    ow = norm(accw_ref)
    oc = norm(accc_ref)
    owb = norm(accwb_ref)
    outw_ref[:] = ow
    outc_ref[:] = oc
    hauth_ref[:] = jnp.dot(owb, Wlin_ref[:],
                           preferred_element_type=jnp.float32) + blin_ref[:]
    kw = jnp.sum(jnp.tanh(jnp.dot(ow, Wk_ref[:],
                                  preferred_element_type=jnp.float32)
                          + bk_ref[:]), axis=0, keepdims=True)
    kc = jnp.sum(jnp.tanh(jnp.dot(oc, Wk_ref[:],
                                  preferred_element_type=jnp.float32)
                          + bk_ref[:]), axis=0, keepdims=True)

    @pl.when(i == 0)
    def _():
        ksw_ref[:] = kw
        ksc_ref[:] = kc

    @pl.when(i > 0)
    def _():
        ksw_ref[:] = ksw_ref[:] + kw
        ksc_ref[:] = ksc_ref[:] + kc


def _post(accw, accc, accwb, Wk, bk, Wlin, blin):
    RB = 200
    grid = (N_NODE // RB,)
    acc_spec = pl.BlockSpec((2, RB, MROW), lambda i: (0, i, 0))
    return pl.pallas_call(
        _post_body,
        grid=grid,
        in_specs=[acc_spec, acc_spec, acc_spec,
                  pl.BlockSpec((HIDDEN, HIDDEN), lambda i: (0, 0)),
                  pl.BlockSpec((1, HIDDEN), lambda i: (0, 0)),
                  pl.BlockSpec((HIDDEN, OUT), lambda i: (0, 0)),
                  pl.BlockSpec((1, OUT), lambda i: (0, 0))],
        out_specs=(pl.BlockSpec((RB, HIDDEN), lambda i: (i, 0)),
                   pl.BlockSpec((RB, HIDDEN), lambda i: (i, 0)),
                   pl.BlockSpec((RB, OUT), lambda i: (i, 0)),
                   pl.BlockSpec((1, HIDDEN), lambda i: (0, 0)),
                   pl.BlockSpec((1, HIDDEN), lambda i: (0, 0))),
        out_shape=(jax.ShapeDtypeStruct((N_NODE, HIDDEN), jnp.float32),
                   jax.ShapeDtypeStruct((N_NODE, HIDDEN), jnp.float32),
                   jax.ShapeDtypeStruct((N_NODE, OUT), jnp.float32),
                   jax.ShapeDtypeStruct((1, HIDDEN), jnp.float32),
                   jax.ShapeDtypeStruct((1, HIDDEN), jnp.float32)),
    )(accw, accc, accwb, Wk, bk, Wlin, blin)


def _final_body(outw_ref, outc_ref, ksw_ref, ksc_ref, q_ref, Wlin_ref,
                blin_ref, hpaper_ref):
    sw = jnp.sum(q_ref[:] * ksw_ref[:]) / N_NODE
    sc = jnp.sum(q_ref[:] * ksc_ref[:]) / N_NODE
    m = jnp.maximum(sw, sc)
    ew = jnp.exp(sw - m)
    ec = jnp.exp(sc - m)
    tot = ew + ec
    h = (ew / tot) * outw_ref[:] + (ec / tot) * outc_ref[:]
    hpaper_ref[:] = jnp.dot(h, Wlin_ref[:],
                            preferred_element_type=jnp.float32) + blin_ref[:]


def _final(outw, outc, ksw, ksc, q, Wlin, blin):
    RB = 200
    grid = (N_NODE // RB,)
    return pl.pallas_call(
        _final_body,
        grid=grid,
        in_specs=[pl.BlockSpec((RB, HIDDEN), lambda i: (i, 0)),
                  pl.BlockSpec((RB, HIDDEN), lambda i: (i, 0)),
                  pl.BlockSpec((1, HIDDEN), lambda i: (0, 0)),
                  pl.BlockSpec((1, HIDDEN), lambda i: (0, 0)),
                  pl.BlockSpec((1, HIDDEN), lambda i: (0, 0)),
                  pl.BlockSpec((HIDDEN, OUT), lambda i: (0, 0)),
                  pl.BlockSpec((1, OUT), lambda i: (0, 0))],
        out_specs=pl.BlockSpec((RB, OUT), lambda i: (i, 0)),
        out_shape=jax.ShapeDtypeStruct((N_NODE, OUT), jnp.float32),
    )(outw, outc, ksw, ksc, q, Wlin, blin)


def _pad_edges(ei):
    pad = EP - E
    src = jnp.concatenate([ei[0], jnp.zeros((pad,), jnp.int32)])
    dst = jnp.concatenate([ei[1], jnp.full((pad,), N_NODE, jnp.int32)])
    return src, dst


def kernel(x_author, x_paper, Wp_author, bp_author, Wp_paper, bp_paper,
           as_writes, ad_writes, as_wb, ad_wb, as_cites, ad_cites,
           Wk, bk, q, Wlin, blin, ei_writes, ei_written_by, ei_cites):
    xa_pad = jnp.pad(x_author, ((0, NACC - N_NODE), (0, 0)))
    xp_pad = jnp.pad(x_paper, ((0, NACC - N_NODE), (0, 0)))
    avecs = tuple(a.reshape(1, HIDDEN) for a in
                  (as_writes, ad_writes, as_wb, ad_wb, as_cites, ad_cites))
    Xa, Xp, Sw, Dw, Swb, Dwb, Sc, Dc = _dense_prep(
        xa_pad, xp_pad, Wp_author, Wp_paper,
        bp_author.reshape(1, HIDDEN), bp_paper.reshape(1, HIDDEN), avecs)

    zeros = jnp.zeros((NSEG, MROW), jnp.float32)
    sw_, dw_ = _pad_edges(ei_writes)
    swb_, dwb_ = _pad_edges(ei_written_by)
    sc_, dc_ = _pad_edges(ei_cites)
    accw = _edge_conv(sw_, dw_, Xa, Sw, Dw, zeros)
    accc = _edge_conv(sc_, dc_, Xp, Sc, Dc, zeros)
    accwb = _edge_conv(swb_, dwb_, Xp, Swb, Dwb, zeros)

    outw, outc, h_author, ksw, ksc = _post(
        accw, accc, accwb, Wk, bk.reshape(1, HIDDEN), Wlin,
        blin.reshape(1, OUT))
    h_paper = _final(outw, outc, ksw, ksc, q.reshape(1, HIDDEN), Wlin,
                     blin.reshape(1, OUT))
    return (h_author, h_paper)


# speedup vs baseline: 76.7397x; 2.3659x over previous
import jax
import jax.numpy as jnp
from jax import lax
from jax.experimental import pallas as pl
from jax.experimental.pallas import tpu as pltpu
from jax.experimental.pallas import tpu_sc as plsc

N_NODE = 10000
D_IN = 256
HIDDEN = 128
HEADS = 8
HEAD_DIM = 16
OUT = 64
E = 160000
NEG_SLOPE = 0.2

NACC = 10240
NSEG = 10016
CHUNK = 120
CHUNKS_PER_TILE = 42
EP = 32 * CHUNKS_PER_TILE * CHUNK
ROWS_PER_TILE = NSEG // 16
MROW = HIDDEN + 16


def _head_expand_mat(cols):
    r = lax.broadcasted_iota(jnp.int32, (HIDDEN, cols), 0)
    c = lax.broadcasted_iota(jnp.int32, (HIDDEN, cols), 1)
    return ((r // HEAD_DIM) == c).astype(jnp.float32)


def _dense_prep_body(xa_ref, xp_ref, Wpa_ref, Wpp_ref, bpa_ref, bpp_ref,
                     asw_ref, adw_ref, aswb_ref, adwb_ref, asc_ref, adc_ref,
                     XSw_ref, XSwb_ref, XSc_ref, Dw_ref, Dwb_ref, Dc_ref):
    xa = jnp.dot(xa_ref[:], Wpa_ref[:], preferred_element_type=jnp.float32)
    xa = xa + bpa_ref[:]
    xp = jnp.dot(xp_ref[:], Wpp_ref[:], preferred_element_type=jnp.float32)
    xp = xp + bpp_ref[:]
    G = _head_expand_mat(16)
    XSw_ref[:, :HIDDEN] = xa
    XSw_ref[:, HIDDEN:] = jnp.dot(xa * asw_ref[:], G,
                                  preferred_element_type=jnp.float32)
    XSwb_ref[:, :HIDDEN] = xp
    XSwb_ref[:, HIDDEN:] = jnp.dot(xp * aswb_ref[:], G,
                                   preferred_element_type=jnp.float32)
    XSc_ref[:, :HIDDEN] = xp
    XSc_ref[:, HIDDEN:] = jnp.dot(xp * asc_ref[:], G,
                                  preferred_element_type=jnp.float32)
    Dw_ref[:] = jnp.dot(xp * adw_ref[:], G, preferred_element_type=jnp.float32)
    Dwb_ref[:] = jnp.dot(xa * adwb_ref[:], G,
                         preferred_element_type=jnp.float32)
    Dc_ref[:] = jnp.dot(xp * adc_ref[:], G, preferred_element_type=jnp.float32)


def _dense_prep(xa_pad, xp_pad, Wpa, Wpp, bpa, bpp, avecs):
    RB = 256
    grid = (NACC // RB,)
    full128 = pl.BlockSpec((D_IN, HIDDEN), lambda i: (0, 0))
    row128 = pl.BlockSpec((1, HIDDEN), lambda i: (0, 0))
    xblk = pl.BlockSpec((RB, D_IN), lambda i: (i, 0))
    o144 = pl.BlockSpec((RB, MROW), lambda i: (i, 0))
    o16 = pl.BlockSpec((RB, 16), lambda i: (i, 0))
    out_shapes = (
        tuple(jax.ShapeDtypeStruct((NACC, MROW), jnp.float32)
              for _ in range(3))
        + tuple(jax.ShapeDtypeStruct((NACC, 16), jnp.float32)
                for _ in range(3)))
    return pl.pallas_call(
        _dense_prep_body,
        grid=grid,
        in_specs=[xblk, xblk, full128, full128, row128, row128] + [row128] * 6,
        out_specs=(o144,) * 3 + (o16,) * 3,
        out_shape=out_shapes,
    )(xa_pad, xp_pad, Wpa, Wpp, bpa, bpp, *avecs)


def _edge_conv_body(si_hbm, di_hbm, XS_hbm, D_hbm, z_hbm, out_hbm,
                    si0, si1, di0, di1, di2, xs0, xs1, dr0, dr1, acc,
                    isem0, isem1, gsem0, gsem1, ssem0, ssem1):
    c = lax.axis_index("c")
    s = lax.axis_index("s")
    wid = c * 16 + s
    sib = [si0, si1]
    dib = [di0, di1, di2]
    xsb = [xs0, xs1]
    drb = [dr0, dr1]
    isems = [isem0, isem1]
    gsems = [gsem0, gsem1]
    ssems = [ssem0, ssem1]

    pltpu.sync_copy(z_hbm.at[pl.ds(s * ROWS_PER_TILE, ROWS_PER_TILE)],
                    acc.at[pl.ds(s * ROWS_PER_TILE, ROWS_PER_TILE)])
    plsc.subcore_barrier()

    idxd, gd, sd = {}, {}, {}

    def fire_idx(j):
        base = (wid * CHUNKS_PER_TILE + j) * CHUNK
        a = pltpu.async_copy(si_hbm.at[pl.ds(base, CHUNK)], sib[j % 2],
                             isems[j % 2])
        b = pltpu.async_copy(di_hbm.at[pl.ds(base, CHUNK)], dib[j % 3],
                             isems[j % 2])
        idxd[j] = (a, b)

    def fire_gather(j):
        p = j % 2
        a = pltpu.async_copy(XS_hbm.at[sib[p]], xsb[p], gsems[p])
        b = pltpu.async_copy(D_hbm.at[dib[j % 3]], drb[p], gsems[p])
        gd[j] = (a, b)

    def compute(j):
        p = j % 2
        xs = xsb[p]
        drows = drb[p]

        def edge(e, carry):
            a = xs[e, pl.ds(HIDDEN, 16)] + drows[e, :]
            a = jnp.where(a > 0.0, a, NEG_SLOPE * a)
            w = jnp.exp(a)
            xs[e, pl.ds(HIDDEN, 16)] = w
            dn = lax.GatherDimensionNumbers(offset_dims=(),
                                            collapsed_slice_dims=(0,),
                                            start_index_map=(0,))
            for h in range(HEADS):
                sh = lax.gather(w, jnp.full((16, 1), h, jnp.int32), dn,
                                slice_sizes=(1,),
                                mode=lax.GatherScatterMode.PROMISE_IN_BOUNDS)
                xs[e, pl.ds(h * 16, 16)] = sh * xs[e, pl.ds(h * 16, 16)]
            return carry

        lax.fori_loop(0, CHUNK, edge, 0)

    fire_idx(0)
    idxd[0][0].wait()
    idxd[0][1].wait()
    fire_gather(0)
    fire_idx(1)

    for j in range(CHUNKS_PER_TILE):
        p = j % 2
        if j >= 1:
            sd[j - 1].wait()
        gd[j][0].wait()
        gd[j][1].wait()
        if j + 1 < CHUNKS_PER_TILE:
            idxd[j + 1][0].wait()
            idxd[j + 1][1].wait()
            fire_gather(j + 1)
        if j + 2 < CHUNKS_PER_TILE:
            fire_idx(j + 2)
        compute(j)
        sd[j] = pltpu.async_copy(xsb[p], acc.at[dib[j % 3]], ssems[p],
                                 add=True)
    sd[CHUNKS_PER_TILE - 1].wait()

    plsc.subcore_barrier()
    pltpu.sync_copy(acc.at[pl.ds(s * ROWS_PER_TILE, ROWS_PER_TILE)],
                    out_hbm.at[c, pl.ds(s * ROWS_PER_TILE, ROWS_PER_TILE)])


def _edge_conv(src_idx, dst_idx, XS, D, zeros):
    mesh = plsc.VectorSubcoreMesh(core_axis_name="c", subcore_axis_name="s",
                                  num_cores=2, num_subcores=16)
    f = pl.kernel(
        _edge_conv_body,
        out_type=jax.ShapeDtypeStruct((2, NSEG, MROW), jnp.float32),
        mesh=mesh,
        scratch_types=[
            pltpu.VMEM((CHUNK,), jnp.int32),
            pltpu.VMEM((CHUNK,), jnp.int32),
            pltpu.VMEM((CHUNK,), jnp.int32),
            pltpu.VMEM((CHUNK,), jnp.int32),
            pltpu.VMEM((CHUNK,), jnp.int32),
            pltpu.VMEM((CHUNK, MROW), jnp.float32),
            pltpu.VMEM((CHUNK, MROW), jnp.float32),
            pltpu.VMEM((CHUNK, 16), jnp.float32),
            pltpu.VMEM((CHUNK, 16), jnp.float32),
            pltpu.VMEM_SHARED((NSEG, MROW), jnp.float32),
            pltpu.SemaphoreType.DMA,
            pltpu.SemaphoreType.DMA,
            pltpu.SemaphoreType.DMA,
            pltpu.SemaphoreType.DMA,
            pltpu.SemaphoreType.DMA,
            pltpu.SemaphoreType.DMA,
        ],
        compiler_params=pltpu.CompilerParams(use_tc_tiling_on_sc=False),
    )
    return f(src_idx, dst_idx, XS, D, zeros)


def _post_body(accw_ref, accc_ref, accwb_ref, Wk_ref, bk_ref, Wlin_ref,
               blin_ref, outw_ref, outc_ref, hauth_ref, ksw_ref, ksc_ref):
    i = pl.program_id(0)
    B = _head_expand_mat(16)

    def norm(acc_ref):
        sblk = acc_ref[0] + acc_ref[1]
        num = sblk[:, :HIDDEN]
        den = sblk[:, HIDDEN:HIDDEN + 16]
        den_b = jnp.dot(den, B.T, preferred_element_type=jnp.float32)
        return jnp.maximum(num / (den_b + 1e-16), 0.0)

    ow = norm(accw_ref)
    oc = norm(accc_ref)
    owb = norm(accwb_ref)
    outw_ref[:] = ow
    outc_ref[:] = oc
    hauth_ref[:] = jnp.dot(owb, Wlin_ref[:],
                           preferred_element_type=jnp.float32) + blin_ref[:]
    kw = jnp.sum(jnp.tanh(jnp.dot(ow, Wk_ref[:],
                                  preferred_element_type=jnp.float32)
                          + bk_ref[:]), axis=0, keepdims=True)
    kc = jnp.sum(jnp.tanh(jnp.dot(oc, Wk_ref[:],
                                  preferred_element_type=jnp.float32)
                          + bk_ref[:]), axis=0, keepdims=True)

    @pl.when(i == 0)
    def _():
        ksw_ref[:] = kw
        ksc_ref[:] = kc

    @pl.when(i > 0)
    def _():
        ksw_ref[:] = ksw_ref[:] + kw
        ksc_ref[:] = ksc_ref[:] + kc


def _post(accw, accc, accwb, Wk, bk, Wlin, blin):
    RB = 200
    grid = (N_NODE // RB,)
    acc_spec = pl.BlockSpec((2, RB, MROW), lambda i: (0, i, 0))
    return pl.pallas_call(
        _post_body,
        grid=grid,
        in_specs=[acc_spec, acc_spec, acc_spec,
                  pl.BlockSpec((HIDDEN, HIDDEN), lambda i: (0, 0)),
                  pl.BlockSpec((1, HIDDEN), lambda i: (0, 0)),
                  pl.BlockSpec((HIDDEN, OUT), lambda i: (0, 0)),
                  pl.BlockSpec((1, OUT), lambda i: (0, 0))],
        out_specs=(pl.BlockSpec((RB, HIDDEN), lambda i: (i, 0)),
                   pl.BlockSpec((RB, HIDDEN), lambda i: (i, 0)),
                   pl.BlockSpec((RB, OUT), lambda i: (i, 0)),
                   pl.BlockSpec((1, HIDDEN), lambda i: (0, 0)),
                   pl.BlockSpec((1, HIDDEN), lambda i: (0, 0))),
        out_shape=(jax.ShapeDtypeStruct((N_NODE, HIDDEN), jnp.float32),
                   jax.ShapeDtypeStruct((N_NODE, HIDDEN), jnp.float32),
                   jax.ShapeDtypeStruct((N_NODE, OUT), jnp.float32),
                   jax.ShapeDtypeStruct((1, HIDDEN), jnp.float32),
                   jax.ShapeDtypeStruct((1, HIDDEN), jnp.float32)),
    )(accw, accc, accwb, Wk, bk, Wlin, blin)


def _final_body(outw_ref, outc_ref, ksw_ref, ksc_ref, q_ref, Wlin_ref,
                blin_ref, hpaper_ref):
    sw = jnp.sum(q_ref[:] * ksw_ref[:]) / N_NODE
    sc = jnp.sum(q_ref[:] * ksc_ref[:]) / N_NODE
    m = jnp.maximum(sw, sc)
    ew = jnp.exp(sw - m)
    ec = jnp.exp(sc - m)
    tot = ew + ec
    h = (ew / tot) * outw_ref[:] + (ec / tot) * outc_ref[:]
    hpaper_ref[:] = jnp.dot(h, Wlin_ref[:],
                            preferred_element_type=jnp.float32) + blin_ref[:]


def _final(outw, outc, ksw, ksc, q, Wlin, blin):
    RB = 200
    grid = (N_NODE // RB,)
    return pl.pallas_call(
        _final_body,
        grid=grid,
        in_specs=[pl.BlockSpec((RB, HIDDEN), lambda i: (i, 0)),
                  pl.BlockSpec((RB, HIDDEN), lambda i: (i, 0)),
                  pl.BlockSpec((1, HIDDEN), lambda i: (0, 0)),
                  pl.BlockSpec((1, HIDDEN), lambda i: (0, 0)),
                  pl.BlockSpec((1, HIDDEN), lambda i: (0, 0)),
                  pl.BlockSpec((HIDDEN, OUT), lambda i: (0, 0)),
                  pl.BlockSpec((1, OUT), lambda i: (0, 0))],
        out_specs=pl.BlockSpec((RB, OUT), lambda i: (i, 0)),
        out_shape=jax.ShapeDtypeStruct((N_NODE, OUT), jnp.float32),
    )(outw, outc, ksw, ksc, q, Wlin, blin)


def _pad_edges(ei):
    pad = EP - E
    src = jnp.concatenate([ei[0], jnp.zeros((pad,), jnp.int32)])
    dst = jnp.concatenate([ei[1], jnp.full((pad,), N_NODE, jnp.int32)])
    return src, dst


def kernel(x_author, x_paper, Wp_author, bp_author, Wp_paper, bp_paper,
           as_writes, ad_writes, as_wb, ad_wb, as_cites, ad_cites,
           Wk, bk, q, Wlin, blin, ei_writes, ei_written_by, ei_cites):
    xa_pad = jnp.pad(x_author, ((0, NACC - N_NODE), (0, 0)))
    xp_pad = jnp.pad(x_paper, ((0, NACC - N_NODE), (0, 0)))
    avecs = tuple(a.reshape(1, HIDDEN) for a in
                  (as_writes, ad_writes, as_wb, ad_wb, as_cites, ad_cites))
    XSw, XSwb, XSc, Dw, Dwb, Dc = _dense_prep(
        xa_pad, xp_pad, Wp_author, Wp_paper,
        bp_author.reshape(1, HIDDEN), bp_paper.reshape(1, HIDDEN), avecs)

    zeros = jnp.zeros((NSEG, MROW), jnp.float32)
    sw_, dw_ = _pad_edges(ei_writes)
    swb_, dwb_ = _pad_edges(ei_written_by)
    sc_, dc_ = _pad_edges(ei_cites)
    accw = _edge_conv(sw_, dw_, XSw, Dw, zeros)
    accc = _edge_conv(sc_, dc_, XSc, Dc, zeros)
    accwb = _edge_conv(swb_, dwb_, XSwb, Dwb, zeros)

    outw, outc, h_author, ksw, ksc = _post(
        accw, accc, accwb, Wk, bk.reshape(1, HIDDEN), Wlin,
        blin.reshape(1, OUT))
    h_paper = _final(outw, outc, ksw, ksc, q.reshape(1, HIDDEN), Wlin,
                     blin.reshape(1, OUT))
    return (h_author, h_paper)
